# baseline (speedup 1.0000x reference)
import numpy as np

import jax
import jax.numpy as jnp
from jax import lax
from jax.experimental import pallas as pl
from jax.experimental.pallas import tpu as pltpu
from jax.experimental.pallas import tpu_sc as plsc

B = 1024
DIM = 128
R = 2
NBR = 16
WPR = DIM // 2

NW = 32
N0 = B * R // NW
N1 = B * R * NBR // NW
N2 = B * R * NBR * NBR // NW
CHUNK = 128
NCH2 = N2 // CHUNK
OUT_PER_CHUNK = CHUNK // NBR

_PERM = np.array([32 * (p // 32) + 2 * (p % 16) + (p % 32) // 16
                  for p in range(DIM)], dtype=np.int32)


def _sc_body(e0_h, e1_h, e2_h, tab_h, tabp_h, v0_h, v1_h, s2_h,
             idx_v, buf_a, buf_b, pbuf_a, pbuf_b, stage, sem_a, sem_b):
    wid = lax.axis_index("s") * 2 + lax.axis_index("c")

    pltpu.sync_copy(e0_h.at[pl.ds(wid * N0, N0)], idx_v.at[pl.ds(0, N0)])
    pltpu.async_copy(tab_h.at[idx_v.at[pl.ds(0, N0)]],
                     buf_a.at[pl.ds(0, N0)], sem_a).wait()
    pltpu.sync_copy(buf_a.at[pl.ds(0, N0)], v0_h.at[pl.ds(wid * N0, N0)])

    pltpu.sync_copy(e1_h.at[pl.ds(wid * N1, N1)], idx_v.at[pl.ds(0, N1)])
    n1ch = N1 // CHUNK
    pend = [
        pltpu.async_copy(tab_h.at[idx_v.at[pl.ds(0, CHUNK)]], buf_a, sem_a),
        pltpu.async_copy(tab_h.at[idx_v.at[pl.ds(CHUNK, CHUNK)]], buf_b, sem_b),
    ]
    for j in range(n1ch):
        pend[j % 2].wait()
        buf = buf_a if j % 2 == 0 else buf_b
        sem = sem_a if j % 2 == 0 else sem_b
        pltpu.sync_copy(buf, v1_h.at[pl.ds(wid * N1 + j * CHUNK, CHUNK)])
        if j + 2 < n1ch:
            pend[j % 2] = pltpu.async_copy(
                tab_h.at[idx_v.at[pl.ds((j + 2) * CHUNK, CHUNK)]], buf, sem)

    pltpu.sync_copy(e2_h.at[pl.ds(wid * N2, N2)], idx_v)

    pltpu.async_copy(tabp_h.at[idx_v.at[pl.ds(0, CHUNK)]], pbuf_a, sem_a)
    pltpu.async_copy(tabp_h.at[idx_v.at[pl.ds(CHUNK, CHUNK)]], pbuf_b, sem_b)

    zero = jnp.zeros((16,), jnp.float32)
    mask_hi = jnp.full((16,), -65536, jnp.int32)
    sh16 = jnp.full((16,), 16, jnp.int32)

    def do_chunk(c, pbuf, sem):
        pltpu.make_async_copy(tabp_h.at[pl.ds(0, CHUNK)], pbuf, sem).wait()

        def obody(o, _):
            base = o * NBR
            accs = [zero] * 8
            for r in range(NBR):
                for t in range(4):
                    x = pbuf[base + r, pl.ds(t * 16, 16)]
                    lo = lax.bitcast_convert_type(
                        lax.shift_left(x, sh16), jnp.float32)
                    hi = lax.bitcast_convert_type(
                        lax.bitwise_and(x, mask_hi), jnp.float32)
                    accs[2 * t] = accs[2 * t] + lo
                    accs[2 * t + 1] = accs[2 * t + 1] + hi
            for p in range(8):
                stage[o, pl.ds(p * 16, 16)] = accs[p]
            return 0

        lax.fori_loop(0, OUT_PER_CHUNK, obody, 0)
        pltpu.sync_copy(
            stage, s2_h.at[pl.ds(wid * (N2 // NBR) + c * OUT_PER_CHUNK,
                                 OUT_PER_CHUNK)])

        @pl.when(c + 2 < NCH2)
        def _():
            pltpu.async_copy(tabp_h.at[idx_v.at[pl.ds((c + 2) * CHUNK, CHUNK)]],
                             pbuf, sem)

    def pair_body(g, _):
        do_chunk(2 * g, pbuf_a, sem_a)
        do_chunk(2 * g + 1, pbuf_b, sem_b)
        return 0

    lax.fori_loop(0, NCH2 // 2, pair_body, 0)


def _sc_gather(e0f, e1f, e2f, table, table_p):
    mesh = plsc.VectorSubcoreMesh(core_axis_name="c", subcore_axis_name="s")
    f = pl.kernel(
        _sc_body,
        out_type=[
            jax.ShapeDtypeStruct((B * R, DIM), jnp.float32),
            jax.ShapeDtypeStruct((B * R * NBR, DIM), jnp.float32),
            jax.ShapeDtypeStruct((B * R * NBR, DIM), jnp.float32),
        ],
        mesh=mesh,
        scratch_types=[
            pltpu.VMEM((N2,), jnp.int32),
            pltpu.VMEM((CHUNK, DIM), jnp.float32),
            pltpu.VMEM((CHUNK, DIM), jnp.float32),
            pltpu.VMEM((CHUNK, WPR), jnp.int32),
            pltpu.VMEM((CHUNK, WPR), jnp.int32),
            pltpu.VMEM((OUT_PER_CHUNK, DIM), jnp.float32),
            pltpu.SemaphoreType.DMA,
            pltpu.SemaphoreType.DMA,
        ],
        compiler_params=pltpu.CompilerParams(use_tc_tiling_on_sc=False),
    )
    return f(e0f, e1f, e2f, table, table_p)


def _tc_body(re_ref, v0_ref, v1_ref, s2_ref, wt_ref, wtp_ref, b_ref, out_ref):
    u = v1_ref.shape[0] // (R * NBR)
    v1 = v1_ref[...]
    s2p = s2_ref[...]
    wt = wt_ref[...]
    bb = b_ref[...]
    prec = lax.Precision.HIGHEST

    h1 = jnp.dot(v1, wt, precision=prec) \
        + jnp.dot(s2p, wtp_ref[...], precision=prec) * (1.0 / NBR) + bb
    h1 = jnp.maximum(h1, 0.0)

    agg1 = jnp.sum(v1.reshape(u * R, NBR, DIM), axis=1) * (1.0 / NBR)
    h0 = jnp.maximum(jnp.dot(v0_ref[...] + agg1, wt, precision=prec) + bb, 0.0)

    aggh1 = jnp.sum(h1.reshape(u * R, NBR, DIM), axis=1) * (1.0 / NBR)
    o = jnp.tanh(jnp.dot(h0 + aggh1, wt, precision=prec) + bb)

    o3 = o.reshape(u, R, DIM)
    r0 = re_ref[0, 0]
    r1 = re_ref[0, 1]
    out_ref[...] = o3[:, 0, :] * r0 + o3[:, 1, :] * r1


def _tc_dense(v0, v1, s2p, wt, wtp, b2, re):
    grid = 8
    u = B // grid
    return pl.pallas_call(
        _tc_body,
        grid=(grid,),
        in_specs=[
            pl.BlockSpec(memory_space=pltpu.SMEM),
            pl.BlockSpec((u * R, DIM), lambda i: (i, 0)),
            pl.BlockSpec((u * R * NBR, DIM), lambda i: (i, 0)),
            pl.BlockSpec((u * R * NBR, DIM), lambda i: (i, 0)),
            pl.BlockSpec((DIM, DIM), lambda i: (0, 0)),
            pl.BlockSpec((DIM, DIM), lambda i: (0, 0)),
            pl.BlockSpec((1, DIM), lambda i: (0, 0)),
        ],
        out_specs=pl.BlockSpec((u, DIM), lambda i: (i, 0)),
        out_shape=jax.ShapeDtypeStruct((B, DIM), jnp.float32),
    )(re, v0, v1, s2p, wt, wtp, b2)


def kernel(e0, e1, e2, ent_embed, rule_embed, W, b):
    tab_p = lax.bitcast_convert_type(
        ent_embed.astype(jnp.bfloat16).reshape(-1, WPR, 2), jnp.int32)
    wt = W.T
    wtp = wt[jnp.asarray(_PERM)]
    v0, v1, s2p = _sc_gather(e0.reshape(-1), e1.reshape(-1), e2.reshape(-1),
                             ent_embed, tab_p)
    return _tc_dense(v0, v1, s2p, wt, wtp, b.reshape(1, DIM), rule_embed)


# speedup vs baseline: 2.8530x; 2.8530x over previous
import jax
import jax.numpy as jnp
from jax import lax
from jax.experimental import pallas as pl
from jax.experimental.pallas import tpu as pltpu
from jax.experimental.pallas import tpu_sc as plsc

B = 1024
DIM = 128
R = 2
NBR = 16

NW = 32
N0 = B * R // NW
N1 = B * R * NBR // NW
N2 = B * R * NBR * NBR // NW
CHUNK = 128
NCH1 = N1 // CHUNK
NCH2 = N2 // CHUNK
OPC = CHUNK // NBR
AG1 = N1 // NBR


def _sc_body(e0_h, e1_h, e2_h, tab_h, v0_h, s1_h, a1_h,
             idx_v, idx1_v, buf_a, buf_b, v1_a, v1_b, stage, astage,
             sem_a, sem_b, sem_va, sem_vb):
    wid = lax.axis_index("s") * 2 + lax.axis_index("c")
    zero = jnp.zeros((16,), jnp.float32)
    inv16 = jnp.full((16,), 1.0 / NBR, jnp.float32)

    pltpu.sync_copy(e0_h.at[pl.ds(wid * N0, N0)], idx1_v.at[pl.ds(0, N0)])
    pltpu.async_copy(tab_h.at[idx1_v.at[pl.ds(0, N0)]],
                     buf_a.at[pl.ds(0, N0)], sem_a).wait()
    pltpu.sync_copy(buf_a.at[pl.ds(0, N0)], v0_h.at[pl.ds(wid * N0, N0)])

    pltpu.sync_copy(e1_h.at[pl.ds(wid * N1, N1)], idx1_v)
    pltpu.sync_copy(e2_h.at[pl.ds(wid * N2, N2)], idx_v)

    pltpu.async_copy(tab_h.at[idx1_v.at[pl.ds(0, CHUNK)]], v1_a, sem_va)
    pltpu.async_copy(tab_h.at[idx1_v.at[pl.ds(CHUNK, CHUNK)]], v1_b, sem_vb)
    pltpu.async_copy(tab_h.at[idx_v.at[pl.ds(0, CHUNK)]], buf_a, sem_a)
    pltpu.async_copy(tab_h.at[idx_v.at[pl.ds(CHUNK, CHUNK)]], buf_b, sem_b)

    def do_chunk(c, t, buf, sem, v1buf):
        pltpu.make_async_copy(tab_h.at[pl.ds(0, CHUNK)], buf, sem).wait()

        def obody(o, _):
            base = o * NBR
            accs = [zero] * 8
            for r in range(NBR):
                for k in range(8):
                    accs[k] = accs[k] + buf[base + r, pl.ds(k * 16, 16)]
            vrow = t * OPC + o
            for k in range(8):
                stage[o, pl.ds(k * 16, 16)] = (
                    v1buf[vrow, pl.ds(k * 16, 16)] + accs[k] * inv16)
            return 0

        lax.fori_loop(0, OPC, obody, 0)
        pltpu.sync_copy(stage, s1_h.at[pl.ds(wid * N1 + c * OPC, OPC)])

        @pl.when(c + 2 < NCH2)
        def _():
            pltpu.async_copy(tab_h.at[idx_v.at[pl.ds((c + 2) * CHUNK, CHUNK)]],
                             buf, sem)

    def run_block(j, v1buf, semv):
        pltpu.make_async_copy(tab_h.at[pl.ds(0, CHUNK)], v1buf, semv).wait()

        def abody(a, _):
            accs = [zero] * 8
            for r in range(NBR):
                for k in range(8):
                    accs[k] = accs[k] + v1buf[a * NBR + r, pl.ds(k * 16, 16)]
            for k in range(8):
                astage[a, pl.ds(k * 16, 16)] = accs[k]
            return 0

        lax.fori_loop(0, OPC, abody, 0)
        pltpu.sync_copy(astage, a1_h.at[pl.ds(wid * AG1 + j * OPC, OPC)])

        def tbody(tt, _):
            t0 = 2 * tt
            do_chunk(16 * j + t0, t0, buf_a, sem_a, v1buf)
            do_chunk(16 * j + t0 + 1, t0 + 1, buf_b, sem_b, v1buf)
            return 0

        lax.fori_loop(0, 8, tbody, 0)

        @pl.when(j + 2 < NCH1)
        def _():
            pltpu.async_copy(
                tab_h.at[idx1_v.at[pl.ds((j + 2) * CHUNK, CHUNK)]], v1buf, semv)

    def qbody(q, _):
        run_block(2 * q, v1_a, sem_va)
        run_block(2 * q + 1, v1_b, sem_vb)
        return 0

    lax.fori_loop(0, NCH1 // 2, qbody, 0)


def _sc_gather(e0f, e1f, e2f, table):
    mesh = plsc.VectorSubcoreMesh(core_axis_name="c", subcore_axis_name="s")
    f = pl.kernel(
        _sc_body,
        out_type=[
            jax.ShapeDtypeStruct((B * R, DIM), jnp.float32),
            jax.ShapeDtypeStruct((B * R * NBR, DIM), jnp.float32),
            jax.ShapeDtypeStruct((B * R, DIM), jnp.float32),
        ],
        mesh=mesh,
        scratch_types=[
            pltpu.VMEM((N2,), jnp.int32),
            pltpu.VMEM((N1,), jnp.int32),
            pltpu.VMEM((CHUNK, DIM), jnp.float32),
            pltpu.VMEM((CHUNK, DIM), jnp.float32),
            pltpu.VMEM((CHUNK, DIM), jnp.float32),
            pltpu.VMEM((CHUNK, DIM), jnp.float32),
            pltpu.VMEM((OPC, DIM), jnp.float32),
            pltpu.VMEM((OPC, DIM), jnp.float32),
            pltpu.SemaphoreType.DMA,
            pltpu.SemaphoreType.DMA,
            pltpu.SemaphoreType.DMA,
            pltpu.SemaphoreType.DMA,
        ],
    )
    return f(e0f, e1f, e2f, table)


def _matmul_t(x, w, prec):
    return lax.dot_general(x, w, (((1,), (1,)), ((), ())), precision=prec)


def _tc_body(re_ref, v0_ref, s1_ref, a1_ref, w_ref, b_ref, out_ref):
    u = s1_ref.shape[0] // (R * NBR)
    s1 = s1_ref[...]
    w = w_ref[...]
    bb = b_ref[...]
    prec = lax.Precision.HIGHEST

    h1 = jnp.maximum(_matmul_t(s1, w, prec) + bb, 0.0)

    h0 = jnp.maximum(
        _matmul_t(v0_ref[...] + a1_ref[...] * (1.0 / NBR), w, prec) + bb, 0.0)

    aggh1 = jnp.sum(h1.reshape(u * R, NBR, DIM), axis=1) * (1.0 / NBR)
    o = jnp.tanh(_matmul_t(h0 + aggh1, w, prec) + bb)

    o3 = o.reshape(u, R, DIM)
    r0 = re_ref[0, 0]
    r1 = re_ref[0, 1]
    out_ref[...] = o3[:, 0, :] * r0 + o3[:, 1, :] * r1


def _tc_dense(v0, s1, a1, W, b2, re):
    grid = 8
    u = B // grid
    return pl.pallas_call(
        _tc_body,
        grid=(grid,),
        in_specs=[
            pl.BlockSpec(memory_space=pltpu.SMEM),
            pl.BlockSpec((u * R, DIM), lambda i: (i, 0)),
            pl.BlockSpec((u * R * NBR, DIM), lambda i: (i, 0)),
            pl.BlockSpec((u * R, DIM), lambda i: (i, 0)),
            pl.BlockSpec((DIM, DIM), lambda i: (0, 0)),
            pl.BlockSpec((1, DIM), lambda i: (0, 0)),
        ],
        out_specs=pl.BlockSpec((u, DIM), lambda i: (i, 0)),
        out_shape=jax.ShapeDtypeStruct((B, DIM), jnp.float32),
    )(re, v0, s1, a1, W, b2)


def kernel(e0, e1, e2, ent_embed, rule_embed, W, b):
    v0, s1, a1 = _sc_gather(e0.reshape(-1), e1.reshape(-1), e2.reshape(-1),
                            ent_embed)
    return _tc_dense(v0, s1, a1, W, b.reshape(1, DIM), rule_embed)


# speedup vs baseline: 4.0799x; 1.4300x over previous
import jax
import jax.numpy as jnp
from jax import lax
from jax.experimental import pallas as pl
from jax.experimental.pallas import tpu as pltpu
from jax.experimental.pallas import tpu_sc as plsc

B = 1024
DIM = 128
R = 2
NBR = 16

NW = 32
N0 = B * R // NW
N1 = B * R * NBR // NW
N2 = B * R * NBR * NBR // NW
CHUNK = 128
NCH1 = N1 // CHUNK
NCH2 = N2 // CHUNK
OPC = CHUNK // NBR


def _sc_body(e0_h, e1_h, e2_h, tab_h, v0_h, v1_h, s2_h,
             idx_v, buf_a, buf_b, buf_c, buf_d, stage_a, stage_b,
             sem_a, sem_b, sem_c, sem_d, fsem_a, fsem_b):
    wid = lax.axis_index("s") * 2 + lax.axis_index("c")
    zero = jnp.zeros((16,), jnp.float32)

    bufs = [buf_a, buf_b, buf_c, buf_d]
    sems = [sem_a, sem_b, sem_c, sem_d]

    pltpu.sync_copy(e0_h.at[pl.ds(wid * N0, N0)], idx_v.at[pl.ds(0, N0)])
    pltpu.async_copy(tab_h.at[idx_v.at[pl.ds(0, N0)]],
                     buf_a.at[pl.ds(0, N0)], sem_a).wait()
    pltpu.sync_copy(buf_a.at[pl.ds(0, N0)], v0_h.at[pl.ds(wid * N0, N0)])

    pltpu.sync_copy(e1_h.at[pl.ds(wid * N1, N1)], idx_v.at[pl.ds(0, N1)])
    pend = [pltpu.async_copy(tab_h.at[idx_v.at[pl.ds(j * CHUNK, CHUNK)]],
                             bufs[j], sems[j]) for j in range(4)]
    for j in range(NCH1):
        pend[j % 4].wait()
        pltpu.sync_copy(bufs[j % 4],
                        v1_h.at[pl.ds(wid * N1 + j * CHUNK, CHUNK)])
        if j + 4 < NCH1:
            pend[j % 4] = pltpu.async_copy(
                tab_h.at[idx_v.at[pl.ds((j + 4) * CHUNK, CHUNK)]],
                bufs[j % 4], sems[j % 4])

    pltpu.sync_copy(e2_h.at[pl.ds(wid * N2, N2)], idx_v)
    for j in range(4):
        pltpu.async_copy(tab_h.at[idx_v.at[pl.ds(j * CHUNK, CHUNK)]],
                         bufs[j], sems[j])

    def do_chunk(c, buf, sem, stage, fsem):
        pltpu.make_async_copy(tab_h.at[pl.ds(0, CHUNK)], buf, sem).wait()

        @pl.when(c >= 2)
        def _():
            pltpu.make_async_copy(stage, s2_h.at[pl.ds(0, OPC)], fsem).wait()

        def obody(o, _):
            base = o * NBR
            accs = [zero] * 8
            for r in range(NBR):
                for k in range(8):
                    accs[k] = accs[k] + buf[base + r, pl.ds(k * 16, 16)]
            for k in range(8):
                stage[o, pl.ds(k * 16, 16)] = accs[k]
            return 0

        lax.fori_loop(0, OPC, obody, 0)
        pltpu.async_copy(stage, s2_h.at[pl.ds(wid * (N2 // NBR) + c * OPC,
                                              OPC)], fsem)

        @pl.when(c + 4 < NCH2)
        def _():
            pltpu.async_copy(tab_h.at[idx_v.at[pl.ds((c + 4) * CHUNK, CHUNK)]],
                             buf, sem)

    def qbody(g, _):
        c0 = 4 * g
        do_chunk(c0, buf_a, sem_a, stage_a, fsem_a)
        do_chunk(c0 + 1, buf_b, sem_b, stage_b, fsem_b)
        do_chunk(c0 + 2, buf_c, sem_c, stage_a, fsem_a)
        do_chunk(c0 + 3, buf_d, sem_d, stage_b, fsem_b)
        return 0

    lax.fori_loop(0, NCH2 // 4, qbody, 0)

    pltpu.make_async_copy(stage_a, s2_h.at[pl.ds(0, OPC)], fsem_a).wait()
    pltpu.make_async_copy(stage_b, s2_h.at[pl.ds(0, OPC)], fsem_b).wait()


def _sc_gather(e0f, e1f, e2f, table):
    mesh = plsc.VectorSubcoreMesh(core_axis_name="c", subcore_axis_name="s")
    f = pl.kernel(
        _sc_body,
        out_type=[
            jax.ShapeDtypeStruct((B * R, DIM), jnp.float32),
            jax.ShapeDtypeStruct((B * R * NBR, DIM), jnp.float32),
            jax.ShapeDtypeStruct((B * R * NBR, DIM), jnp.float32),
        ],
        mesh=mesh,
        scratch_types=[
            pltpu.VMEM((N2,), jnp.int32),
            pltpu.VMEM((CHUNK, DIM), jnp.float32),
            pltpu.VMEM((CHUNK, DIM), jnp.float32),
            pltpu.VMEM((CHUNK, DIM), jnp.float32),
            pltpu.VMEM((CHUNK, DIM), jnp.float32),
            pltpu.VMEM((OPC, DIM), jnp.float32),
            pltpu.VMEM((OPC, DIM), jnp.float32),
            pltpu.SemaphoreType.DMA,
            pltpu.SemaphoreType.DMA,
            pltpu.SemaphoreType.DMA,
            pltpu.SemaphoreType.DMA,
            pltpu.SemaphoreType.DMA,
            pltpu.SemaphoreType.DMA,
        ],
    )
    return f(e0f, e1f, e2f, table)


def _matmul_t(x, w, prec):
    return lax.dot_general(x, w, (((1,), (1,)), ((), ())), precision=prec)


def _tc_body(re_ref, v0_ref, v1_ref, s2_ref, w_ref, b_ref, out_ref):
    u = v1_ref.shape[0] // (R * NBR)
    v1 = v1_ref[...]
    w = w_ref[...]
    bb = b_ref[...]
    prec = lax.Precision.HIGHEST

    s1 = v1 + s2_ref[...] * (1.0 / NBR)
    h1 = jnp.maximum(_matmul_t(s1, w, prec) + bb, 0.0)

    agg1 = jnp.sum(v1.reshape(u * R, NBR, DIM), axis=1) * (1.0 / NBR)
    h0 = jnp.maximum(_matmul_t(v0_ref[...] + agg1, w, prec) + bb, 0.0)

    aggh1 = jnp.sum(h1.reshape(u * R, NBR, DIM), axis=1) * (1.0 / NBR)
    o = jnp.tanh(_matmul_t(h0 + aggh1, w, prec) + bb)

    o3 = o.reshape(u, R, DIM)
    r0 = re_ref[0, 0]
    r1 = re_ref[0, 1]
    out_ref[...] = o3[:, 0, :] * r0 + o3[:, 1, :] * r1


def _tc_dense(v0, v1, s2, W, b2, re):
    grid = 8
    u = B // grid
    return pl.pallas_call(
        _tc_body,
        grid=(grid,),
        in_specs=[
            pl.BlockSpec(memory_space=pltpu.SMEM),
            pl.BlockSpec((u * R, DIM), lambda i: (i, 0)),
            pl.BlockSpec((u * R * NBR, DIM), lambda i: (i, 0)),
            pl.BlockSpec((u * R * NBR, DIM), lambda i: (i, 0)),
            pl.BlockSpec((DIM, DIM), lambda i: (0, 0)),
            pl.BlockSpec((1, DIM), lambda i: (0, 0)),
        ],
        out_specs=pl.BlockSpec((u, DIM), lambda i: (i, 0)),
        out_shape=jax.ShapeDtypeStruct((B, DIM), jnp.float32),
    )(re, v0, v1, s2, W, b2)


def kernel(e0, e1, e2, ent_embed, rule_embed, W, b):
    v0, v1, s2 = _sc_gather(e0.reshape(-1), e1.reshape(-1), e2.reshape(-1),
                            ent_embed)
    return _tc_dense(v0, v1, s2, W, b.reshape(1, DIM), rule_embed)
